# Initial kernel scaffold; baseline (speedup 1.0000x reference)
import jax
import jax.numpy as jnp
from jax.experimental import pallas as pl


def kernel(X, A, hidden, W_z1, W_z2, W_r1, W_r2, W_h1, W_h2, bias_z, bias_r, bias_h):
    raise NotImplementedError("write your pallas kernel here")


# speedup vs baseline: 1.3210x; 1.3210x over previous
import functools

import jax
import jax.numpy as jnp
from jax.experimental import pallas as pl
from jax.experimental.pallas import tpu as pltpu

B, N, XD, H = 4, 4096, 64, 32
TI = 256
TK = 1024


def _body(a_ref, xh_ref, wb_ref, hid_ref, bz_ref, br_ref, bh_ref,
          out_ref, m_scr, acc_ref):
    i = pl.program_id(0)
    k = pl.program_id(1)

    @pl.when(i == 0)
    def _():
        wb = wb_ref[...]
        for b in range(B):
            m_scr[b, pl.ds(k * TK, TK), :] = jnp.dot(
                xh_ref[b], wb, preferred_element_type=jnp.float32)

    @pl.when(k == 0)
    def _():
        acc_ref[...] = jnp.zeros_like(acc_ref)

    a = a_ref[...]
    for b in range(B):
        acc_ref[b] += jnp.dot(a, m_scr[b, pl.ds(k * TK, TK), :],
                              preferred_element_type=jnp.float32)

    @pl.when(k == pl.num_programs(1) - 1)
    def _():
        bz = bz_ref[...]
        br = br_ref[...]
        bh = bh_ref[...]
        for b in range(B):
            c = acc_ref[b]
            z = jax.nn.sigmoid(c[:, 0:H] + bz)
            r = jax.nn.sigmoid(c[:, H:2 * H] + br)
            hv = jnp.tanh(c[:, 2 * H:3 * H] + r * c[:, 3 * H:4 * H] + bh)
            out_ref[b] = z * hid_ref[b] + (1.0 - z) * hv


@functools.partial(jax.jit, static_argnames=("interpret",))
def _run(X, A, hidden, W_z1, W_z2, W_r1, W_r2, W_h1, W_h2,
         bias_z, bias_r, bias_h, interpret=False):
    f32 = jnp.float32
    zeros_col = jnp.zeros((B, N, 2 * H), dtype=f32)
    XH = jnp.concatenate([X, hidden, zeros_col], axis=-1)
    top = jnp.concatenate(
        [W_z1, W_r1, W_h1, jnp.zeros((XD, H), f32)], axis=1)
    mid = jnp.concatenate(
        [W_z2, W_r2, jnp.zeros((H, H), f32), W_h2], axis=1)
    bot = jnp.zeros((H, 4 * H), f32)
    W_big = jnp.concatenate([top, mid, bot], axis=0)

    num_i = N // TI
    num_k = N // TK

    def xh_index(i, k):
        return (0, jnp.where(i == 0, k, 0), 0)

    in_specs = [
            pl.BlockSpec((TI, TK), lambda i, k: (i, k)),
            pl.BlockSpec((B, TK, 4 * H), xh_index),
            pl.BlockSpec((4 * H, 4 * H), lambda i, k: (0, 0)),
            pl.BlockSpec((B, TI, H), lambda i, k: (0, i, 0)),
            pl.BlockSpec((TI, H), lambda i, k: (i, 0)),
            pl.BlockSpec((TI, H), lambda i, k: (i, 0)),
            pl.BlockSpec((TI, H), lambda i, k: (i, 0)),
    ]

    return pl.pallas_call(
        _body,
        grid=(num_i, num_k),
        in_specs=in_specs,
        out_specs=pl.BlockSpec((B, TI, H), lambda i, k: (0, i, 0)),
        out_shape=jax.ShapeDtypeStruct((B, N, H), f32),
        scratch_shapes=[
            pltpu.VMEM((B, N, 4 * H), f32),
            pltpu.VMEM((B, TI, 4 * H), f32),
        ],
        compiler_params=pltpu.CompilerParams(
            dimension_semantics=("arbitrary", "arbitrary"),
        ),
        interpret=interpret,
    )(A, XH, W_big, hidden, bias_z, bias_r, bias_h)


def kernel(X, A, hidden, W_z1, W_z2, W_r1, W_r2, W_h1, W_h2,
           bias_z, bias_r, bias_h):
    return _run(X, A, hidden, W_z1, W_z2, W_r1, W_r2, W_h1, W_h2,
                bias_z, bias_r, bias_h)
